# baseline (speedup 1.0000x reference)
import functools
import jax
import jax.numpy as jnp
from jax import lax
from jax.experimental import pallas as pl
from jax.experimental.pallas import tpu as pltpu
from jax.experimental.pallas import tpu_sc as plsc

_N, _C = 32768, 100
_NW = 32
_RW = _N // _NW
_CH = 128
_NCH = _RW // _CH
_K = 7


def _sc_body(x_hbm, wb_hbm, b_hbm, out_hbm, xv0, xv1, wv, bv, ov, sem0, sem1, osem):
    wid = lax.axis_index("s") * 2 + lax.axis_index("c")
    base = wid * _RW

    pltpu.sync_copy(wb_hbm, wv)
    pltpu.sync_copy(b_hbm, bv)
    biasvec = bv[...]
    lanes = lax.iota(jnp.int32, 16)
    perms = {h: lanes ^ h for h in (8, 4, 2, 1)}
    masks = {h: (lanes & h) == 0 for h in (8, 4, 2, 1)}

    bufs = (xv0, xv1)
    sems = (sem0, sem1)
    copies = [
        pltpu.async_copy(x_hbm.at[pl.ds(base, _CH)], xv0, sem0),
        pltpu.async_copy(x_hbm.at[pl.ds(base + _CH, _CH)], xv1, sem1),
    ]

    wk = [wv[k] for k in range(_K)]

    bitrev = (0, 8, 4, 12, 2, 10, 6, 14, 1, 9, 5, 13, 3, 11, 7, 15)

    def _shuf(v, h):
        return v.at[perms[h]].get(mode="promise_in_bounds")

    def do_chunk(ci, xv):
        def group_body(g, carry):
            r0 = g * 16
            accs = [None] * 16
            for l in range(16):
                r = r0 + l
                t = xv[r, pl.ds(0, 16)] * wk[0]
                acc = jnp.where(t != t, jnp.float32(0.0), t)
                for k in range(1, 5):
                    t = xv[r, pl.ds(16 * k, 16)] * wk[k]
                    acc = acc + jnp.where(t != t, jnp.float32(0.0), t)
                t = xv[r, pl.ds(80, 16)] * wk[5]
                acc = acc + jnp.where(t != t, jnp.float32(0.0), t)
                t = xv[r, pl.ds(84, 16)] * wk[6]
                acc = acc + jnp.where(t != t, jnp.float32(0.0), t)
                accs[l] = acc
            vs = [accs[p] for p in bitrev]
            for h in (8, 4, 2, 1):
                nxt_vs = []
                for i in range(0, len(vs), 2):
                    fu = vs[i] + _shuf(vs[i], h)
                    fv = vs[i + 1] + _shuf(vs[i + 1], h)
                    nxt_vs.append(jnp.where(masks[h], fu, fv))
                vs = nxt_vs
            ov[pl.ds(r0, 16)] = vs[0] + biasvec
            return carry

        lax.fori_loop(0, _CH // 16, group_body, 0)
        pltpu.async_copy(ov, out_hbm.at[pl.ds(base + ci * _CH, _CH)], osem).wait()

    for ci in range(_NCH):
        b = ci % 2
        copies[b].wait()
        do_chunk(ci, bufs[b])
        nxt = ci + 2
        if nxt < _NCH:
            copies[b] = pltpu.async_copy(
                x_hbm.at[pl.ds(base + nxt * _CH, _CH)], bufs[b], sems[b]
            )


def kernel(local_map_predictions, weights_pool, bias):
    x = local_map_predictions
    w = weights_pool
    wb = jnp.zeros((_K, 16), jnp.float32)
    wb = wb.at[0:5, :].set(w[:80].reshape(5, 16))
    wb = wb.at[5, :].set(w[80:96])
    wb = wb.at[6, 12:].set(w[96:100])

    mesh = plsc.VectorSubcoreMesh(core_axis_name="c", subcore_axis_name="s")
    run = functools.partial(
        pl.kernel,
        mesh=mesh,
        out_type=jax.ShapeDtypeStruct((_N,), jnp.float32),
        scratch_types=[
            pltpu.VMEM((_CH, _C), jnp.float32),
            pltpu.VMEM((_CH, _C), jnp.float32),
            pltpu.VMEM((_K, 16), jnp.float32),
            pltpu.VMEM((16,), jnp.float32),
            pltpu.VMEM((_CH,), jnp.float32),
            pltpu.SemaphoreType.DMA,
            pltpu.SemaphoreType.DMA,
            pltpu.SemaphoreType.DMA,
        ],
    )(_sc_body)
    b16 = jnp.full((16,), bias[0], jnp.float32)
    out = run(x, wb, b16)
    return out[:, None]


# speedup vs baseline: 4.6116x; 4.6116x over previous
import jax
import jax.numpy as jnp
from jax.experimental import pallas as pl
from jax.experimental.pallas import tpu as pltpu

_N, _C = 32768, 100
_B = 4096


def _tc_body(xt_ref, w_ref, b_ref, o_ref):
    t = xt_ref[...] * w_ref[...]
    contrib = jnp.where(t != t, jnp.float32(0.0), t)
    o_ref[...] = jnp.sum(contrib, axis=0, keepdims=True) + b_ref[0]


def kernel(local_map_predictions, weights_pool, bias):
    xt = jnp.swapaxes(local_map_predictions, 0, 1)
    w2 = weights_pool[:, None]
    out = pl.pallas_call(
        _tc_body,
        grid=(_N // _B,),
        in_specs=[
            pl.BlockSpec((_C, _B), lambda i: (0, i)),
            pl.BlockSpec((_C, 1), lambda i: (0, 0)),
            pl.BlockSpec(memory_space=pltpu.SMEM),
        ],
        out_specs=pl.BlockSpec((1, _B), lambda i: (0, i)),
        out_shape=jax.ShapeDtypeStruct((1, _N), jnp.float32),
    )(xt, w2, bias)
    return jnp.swapaxes(out, 0, 1)


# speedup vs baseline: 4.8364x; 1.0488x over previous
import jax
import jax.numpy as jnp
from jax.experimental import pallas as pl
from jax.experimental.pallas import tpu as pltpu

_N, _C = 32768, 100
_CP = 104
_S = 4
_B = 4096
_G = _N // (_S * _B)


def _tc_body(x0, x1, x2, x3, w_ref, b_ref, o_ref):
    w = w_ref[...]
    b = b_ref[0]
    for k, xr in enumerate((x0, x1, x2, x3)):
        t = xr[...] * w
        contrib = jnp.where(t != t, jnp.float32(0.0), t)
        o_ref[k, :] = jnp.sum(contrib, axis=0) + b


def _mk_spec(k):
    return pl.BlockSpec((_CP, _B), lambda i, k=k: (0, k * _G + i))


def kernel(local_map_predictions, weights_pool, bias):
    xt = jnp.swapaxes(local_map_predictions, 0, 1)
    w2 = jnp.zeros((_CP, 1), jnp.float32).at[:_C, 0].set(weights_pool)
    out = pl.pallas_call(
        _tc_body,
        grid=(_G,),
        in_specs=[
            _mk_spec(0),
            _mk_spec(1),
            _mk_spec(2),
            _mk_spec(3),
            pl.BlockSpec((_CP, 1), lambda i: (0, 0)),
            pl.BlockSpec(memory_space=pltpu.SMEM),
        ],
        out_specs=pl.BlockSpec((_S, _B), lambda i: (0, i)),
        out_shape=jax.ShapeDtypeStruct((_S, _N // _S), jnp.float32),
    )(xt, xt, xt, xt, w2, bias)
    return out.reshape(_N)[:, None]
